# Initial kernel scaffold; baseline (speedup 1.0000x reference)
import jax
import jax.numpy as jnp
from jax.experimental import pallas as pl


def kernel(inputs):
    raise NotImplementedError("write your pallas kernel here")


# speedup vs baseline: 1.0285x; 1.0285x over previous
import functools

import jax
import jax.numpy as jnp
from jax import lax
from jax.experimental import pallas as pl
from jax.experimental.pallas import tpu as pltpu
from jax.experimental.pallas import tpu_sc as plsc

SEQ = 2048
COLS = SEQ + 2
BATCH = 16384
L = 16
NC = 2
NS = 16
NW = NC * NS
ROWS_W = BATCH // NW
R = 8
NBLK = ROWS_W // R

def _build(interpret=False):
    mesh = plsc.VectorSubcoreMesh(
        core_axis_name="c", subcore_axis_name="s", num_cores=NC, num_subcores=NS)
    return functools.partial(
        pl.kernel,
        out_type=jax.ShapeDtypeStruct((BATCH, 3), jnp.float32),
        mesh=mesh,
        scratch_types=[
            pltpu.VMEM((R, COLS), jnp.float32),
            pltpu.VMEM((R, COLS), jnp.float32),
            pltpu.VMEM((ROWS_W, 3), jnp.float32),
            pltpu.VMEM((L,), jnp.float32),
            pltpu.SemaphoreType.DMA,
            pltpu.SemaphoreType.DMA,
        ],
        compiler_params=pltpu.CompilerParams(
            needs_layout_passes=False, use_tc_tiling_on_sc=False),
        interpret=interpret,
    )(_body)


def _body(in_hbm, out_hbm, buf0, buf1, obuf, scr, sem0, sem1):
    c = lax.axis_index("c")
    s = lax.axis_index("s")
    wid = s * NC + c
    base = wid * ROWS_W

    iot = lax.broadcasted_iota(jnp.int32, (L,), 0)
    ones = jnp.ones((L,), jnp.float32)
    out_mask = iot < 3
    is0 = iot == 0
    is1 = iot == 1
    perms = [iot ^ 8, iot ^ 4, iot ^ 2, iot ^ 1]

    def hprod(v):
        for p in perms:
            scr[...] = v
            v = v * plsc.load_gather(scr, [p])
        return v

    def do_rows(buf, blk):
        def row_body(r, _):
            tail = buf[r, pl.ds(COLS - L, L)]
            bid = tail[L - 2].astype(jnp.int32)
            mp = tail[L - 1].astype(jnp.int32)
            nch = (jnp.maximum(bid, mp) + (L - 1)) // L

            def chunk(j, accs):
                ab, am = accs
                v = buf[r, pl.ds(j * L, L)]
                off = j * L
                ab = ab * jnp.where(iot < bid - off, v, ones)
                am = am * jnp.where(iot < mp - off, v, ones)
                return ab, am

            ab, am = lax.fori_loop(0, nch, chunk, (ones, ones))
            sv = hprod(ab)
            pm = hprod(am)
            vmp = plsc.load_gather(
                buf, [jnp.full((L,), r, jnp.int32), jnp.full((L,), mp, jnp.int32)])
            a1 = pm * vmp
            w = jnp.where(is0, sv, jnp.where(is1, a1, pm))
            row = blk * R + r
            plsc.store_scatter(
                obuf, [jnp.full((L,), row, jnp.int32), jnp.minimum(iot, 2)],
                w, mask=out_mask)
            return 0

        lax.fori_loop(0, R, row_body, 0)

    pltpu.async_copy(in_hbm.at[pl.ds(base, R)], buf0, sem0)
    pltpu.async_copy(in_hbm.at[pl.ds(base + R, R)], buf1, sem1)

    def pair_body(p, _):
        b0 = p * 2
        b1 = b0 + 1

        pltpu.make_async_copy(
            in_hbm.at[pl.ds(base + b0 * R, R)], buf0, sem0).wait()
        do_rows(buf0, b0)

        @pl.when(b0 + 2 < NBLK)
        def _():
            pltpu.async_copy(
                in_hbm.at[pl.ds(base + (b0 + 2) * R, R)], buf0, sem0)

        pltpu.make_async_copy(
            in_hbm.at[pl.ds(base + b1 * R, R)], buf1, sem1).wait()
        do_rows(buf1, b1)

        @pl.when(b1 + 2 < NBLK)
        def _():
            pltpu.async_copy(
                in_hbm.at[pl.ds(base + (b1 + 2) * R, R)], buf1, sem1)

        return 0

    lax.fori_loop(0, NBLK // 2, pair_body, 0)

    pltpu.sync_copy(obuf, out_hbm.at[pl.ds(base, ROWS_W)])


_bid_prefix_sc = _build()


def kernel(inputs):
    return _bid_prefix_sc(inputs)


# speedup vs baseline: 5.1133x; 4.9715x over previous
import functools

import jax
import jax.numpy as jnp
from jax import lax
from jax.experimental import pallas as pl
from jax.experimental.pallas import tpu as pltpu
from jax.experimental.pallas import tpu_sc as plsc

SEQ = 2048
COLS = SEQ + 2
BATCH = 16384
L = 16
NC = 2
NS = 16
NW = NC * NS
SAMP_W = BATCH // NW
G = 128
NG = SAMP_W // G
NSG = G // L
PB = 256
NBLK = SEQ // PB


def _build(interpret=False):
    mesh = plsc.VectorSubcoreMesh(
        core_axis_name="c", subcore_axis_name="s", num_cores=NC, num_subcores=NS)
    return functools.partial(
        pl.kernel,
        out_type=jax.ShapeDtypeStruct((3, BATCH), jnp.float32),
        mesh=mesh,
        scratch_types=[
            pltpu.VMEM((PB, G), jnp.float32),
            pltpu.VMEM((PB, G), jnp.float32),
            pltpu.VMEM((8, G), jnp.float32),
            pltpu.VMEM((8, G), jnp.float32),
            pltpu.SemaphoreType.DMA,
            pltpu.SemaphoreType.DMA,
        ],
        compiler_params=pltpu.CompilerParams(
            needs_layout_passes=False, use_tc_tiling_on_sc=True),
        interpret=interpret,
    )(_body)


def _body(xt, out, buf0, buf1, idxb, accb, sem0, sem1):
    c = lax.axis_index("c")
    s = lax.axis_index("s")
    wid = s * NC + c
    base = wid * SAMP_W

    iot = lax.broadcasted_iota(jnp.int32, (L,), 0)
    ones = jnp.ones((L,), jnp.float32)

    def group_body(g, _):
        c0 = base + g * G
        pltpu.sync_copy(xt.at[pl.ds(SEQ, 2), pl.ds(c0, G)],
                        idxb.at[pl.ds(0, 2), :])
        for sg in range(NSG):
            accb[0, pl.ds(sg * L, L)] = ones
            accb[2, pl.ds(sg * L, L)] = ones

        pltpu.async_copy(xt.at[pl.ds(0, PB), pl.ds(c0, G)], buf0, sem0)
        pltpu.async_copy(xt.at[pl.ds(PB, PB), pl.ds(c0, G)], buf1, sem1)

        def do_block(buf, b):
            p0 = b * PB
            for sg in range(NSG):
                cs = sg * L
                bv = idxb[0, pl.ds(cs, L)].astype(jnp.int32) - p0
                mv = idxb[1, pl.ds(cs, L)].astype(jnp.int32) - p0
                ab = accb[0, pl.ds(cs, L)]
                am = accb[2, pl.ds(cs, L)]

                def pos_body(i, accs):
                    ab, am = accs
                    for k in range(8):
                        p = i * 8 + k
                        v = buf[p, pl.ds(cs, L)]
                        ab = ab * jnp.where(bv > p, v, ones)
                        am = am * jnp.where(mv > p, v, ones)
                    return ab, am

                ab, am = lax.fori_loop(0, PB // 8, pos_body, (ab, am))
                accb[0, pl.ds(cs, L)] = ab
                accb[2, pl.ds(cs, L)] = am
                inb = (mv >= 0) & (mv < PB)
                rel = jnp.minimum(jnp.maximum(mv, 0), PB - 1)
                vm = plsc.load_gather(buf, [rel, cs + iot])
                accb[1, pl.ds(cs, L)] = jnp.where(
                    inb, vm, accb[1, pl.ds(cs, L)])

        def pair_body(pb, _):
            b0 = pb * 2
            b1 = b0 + 1
            pltpu.make_async_copy(
                xt.at[pl.ds(b0 * PB, PB), pl.ds(c0, G)], buf0, sem0).wait()
            do_block(buf0, b0)

            @pl.when(b0 + 2 < NBLK)
            def _():
                pltpu.async_copy(
                    xt.at[pl.ds((b0 + 2) * PB, PB), pl.ds(c0, G)], buf0, sem0)

            pltpu.make_async_copy(
                xt.at[pl.ds(b1 * PB, PB), pl.ds(c0, G)], buf1, sem1).wait()
            do_block(buf1, b1)

            @pl.when(b1 + 2 < NBLK)
            def _():
                pltpu.async_copy(
                    xt.at[pl.ds((b1 + 2) * PB, PB), pl.ds(c0, G)], buf1, sem1)

            return 0

        lax.fori_loop(0, NBLK // 2, pair_body, 0)

        for sg in range(NSG):
            cs = sg * L
            accb[1, pl.ds(cs, L)] = accb[1, pl.ds(cs, L)] * accb[2, pl.ds(cs, L)]

        pltpu.sync_copy(accb.at[pl.ds(0, 3), :], out.at[:, pl.ds(c0, G)])
        return 0

    lax.fori_loop(0, NG, group_body, 0)


_bid_prefix_sc = _build()


def kernel(inputs):
    res = _bid_prefix_sc(inputs.T)
    return res.T
